# baseline (speedup 1.0000x reference)
import functools

import jax
import jax.numpy as jnp
from jax import lax
from jax.experimental import pallas as pl
from jax.experimental.pallas import tpu as pltpu
from jax.experimental.pallas import tpu_sc as plsc

N = 10000
D = 128
E = 320000
NC = 2
NS = 16
NW = NC * NS
EPT = E // NW
K = 128
CHI = 79
EPTP = CHI * K
NB = 2
NI = 4
RPT = 640
N_PAD = RPT * NS


def _propagate(x, eidx, zrows):
  mesh = plsc.VectorSubcoreMesh(core_axis_name="c", subcore_axis_name="s")

  @functools.partial(
      pl.kernel,
      out_type=jax.ShapeDtypeStruct((NC, N_PAD, D), jnp.float32),
      mesh=mesh,
      scratch_types=[
          pltpu.VMEM((NI, 2, K), jnp.int32),
          pltpu.VMEM((NB, K, D), jnp.float32),
          pltpu.VMEM_SHARED((N_PAD, D), jnp.float32),
          pltpu.SemaphoreType.DMA((NI,)),
          pltpu.SemaphoreType.DMA((NB,)),
          pltpu.SemaphoreType.DMA((NB,)),
      ],
  )
  def prop(x_hbm, e_hbm, z_hbm, out_hbm, idx_v, rows_v, acc,
           sem_i, sem_g, sem_s):
    cid = lax.axis_index("c")
    sid = lax.axis_index("s")
    wid = sid * NC + cid
    pltpu.sync_copy(z_hbm, acc.at[pl.ds(sid * RPT, RPT)])

    def idx_fetch(j, si):
      pltpu.async_copy(e_hbm.at[wid, j], idx_v.at[si], sem_i.at[si])

    def idx_wait(j, si):
      pltpu.make_async_copy(e_hbm.at[wid, j], idx_v.at[si],
                            sem_i.at[si]).wait()

    def gather(j, b, si):
      pltpu.async_copy(x_hbm.at[idx_v.at[si, 0]], rows_v.at[b],
                       sem_g.at[b])

    def visit(j, b, si, first=False, do_idx=True, do_gather=True):
      bp = (b - 1) % NB
      sip = (si - 1) % NI
      pltpu.make_async_copy(x_hbm.at[idx_v.at[si, 0]], rows_v.at[b],
                            sem_g.at[b]).wait()
      pltpu.async_copy(rows_v.at[b], acc.at[idx_v.at[si, 1]], sem_s.at[b],
                       add=True)
      if not first:
        pltpu.make_async_copy(rows_v.at[bp], acc.at[idx_v.at[sip, 1]],
                              sem_s.at[bp]).wait()
        if do_idx:
          idx_fetch(j - 1 + NI, sip)
        if do_gather:
          jn = j - 1 + NB
          idx_wait(jn, jn % NI)
          gather(jn, bp, jn % NI)

    plsc.subcore_barrier()
    for si in range(NI):
      idx_fetch(si, si)
    for b in range(NB):
      idx_wait(b, b)
      gather(b, b, b)
    for j in range(NI):
      visit(j, j % NB, j % NI, first=j == 0,
            do_idx=j - 1 + NI < CHI, do_gather=j - 1 + NB < CHI)

    G = CHI // NI
    R = CHI - G * NI

    def round_body(g, carry):
      for v in range(NI):
        visit(g * NI + v, v % NB, v % NI)
      return carry

    lax.fori_loop(1, G - 1, round_body, 0)
    for v in range(NI):
      j = (G - 1) * NI + v
      visit(j, v % NB, v % NI,
            do_idx=j - 1 + NI < CHI, do_gather=j - 1 + NB < CHI)
    for v in range(R):
      j = G * NI + v
      visit(j, j % NB, j % NI,
            do_idx=j - 1 + NI < CHI, do_gather=j - 1 + NB < CHI)
    pltpu.make_async_copy(rows_v.at[(CHI - 1) % NB],
                          acc.at[idx_v.at[(CHI - 1) % NI, 1]],
                          sem_s.at[(CHI - 1) % NB]).wait()
    plsc.subcore_barrier()
    pltpu.sync_copy(acc.at[pl.ds(sid * RPT, RPT)],
                    out_hbm.at[cid, pl.ds(sid * RPT, RPT)])

  return prop(x, eidx, zrows)


def _dotT(a, w):
  return lax.dot_general(a, w, (((1,), (1,)), ((), ())),
                         preferred_element_type=jnp.float32)


def _layer_body(x_ref, pa_ref, pb_ref, wl_ref, wr_ref, b_ref, o_ref):
  p = pa_ref[0] + pb_ref[0]
  h = _dotT(x_ref[...], wl_ref[...]) + _dotT(p, wr_ref[...]) + b_ref[...]
  nrm = jnp.sqrt(jnp.sum(h * h, axis=1, keepdims=True))
  h = h / jnp.maximum(nrm, 1e-12)
  o_ref[...] = jnp.maximum(h, 0.0)


def _final_body(x_ref, pa_ref, pb_ref, wl_ref, wr_ref, b_ref,
                wp1_ref, bp1_ref, wp2_ref, bp2_ref, o_ref):
  p = pa_ref[0] + pb_ref[0]
  h = _dotT(x_ref[...], wl_ref[...]) + _dotT(p, wr_ref[...]) + b_ref[...]
  nrm = jnp.sqrt(jnp.sum(h * h, axis=1, keepdims=True))
  h = h / jnp.maximum(nrm, 1e-12)
  h = jnp.maximum(h, 0.0)
  h = _dotT(h, wp1_ref[...]) + bp1_ref[...]
  h = _dotT(h, wp2_ref[...]) + bp2_ref[...]
  m = jnp.max(h, axis=1, keepdims=True)
  lse = m + jnp.log(jnp.sum(jnp.exp(h - m), axis=1, keepdims=True))
  o_ref[...] = h - lse


BR = 1000

_row_spec = pl.BlockSpec((BR, D), lambda i: (i, 0))
_p0_spec = pl.BlockSpec((1, BR, D), lambda i: (0, i, 0))
_p1_spec = pl.BlockSpec((1, BR, D), lambda i: (1, i, 0))
_w_spec = pl.BlockSpec((D, D), lambda i: (0, 0))
_b_spec = pl.BlockSpec((1, D), lambda i: (0, 0))


def _layer_tc(x, P, Wl, Wr, b):
  return pl.pallas_call(
      _layer_body,
      grid=(N // BR,),
      in_specs=[_row_spec, _p0_spec, _p1_spec, _w_spec, _w_spec, _b_spec],
      out_specs=_row_spec,
      out_shape=jax.ShapeDtypeStruct((N, D), jnp.float32),
  )(x, P, P, Wl, Wr, b)


def _final_tc(h, P, Wl, Wr, b, Wp1, bp1, Wp2, bp2):
  return pl.pallas_call(
      _final_body,
      grid=(N // BR,),
      in_specs=[_row_spec, _p0_spec, _p1_spec, _w_spec, _w_spec, _b_spec,
                _w_spec, _b_spec, _w_spec, _b_spec],
      out_specs=_row_spec,
      out_shape=jax.ShapeDtypeStruct((N, D), jnp.float32),
  )(h, P, P, Wl, Wr, b, Wp1, bp1, Wp2, bp2)


def kernel(x, edge_index, batch, Wl0, bl0, Wr0, br0, Wl1, bl1, Wr1, br1,
           Wp1, bp1, Wp2, bp2):
  del batch
  ei = edge_index.reshape(2, NW, EPT)
  pad = ((0, 0), (0, EPTP - EPT))
  srcp = jnp.pad(ei[0], pad, constant_values=0)
  dstp = jnp.pad(ei[1], pad, constant_values=N_PAD - 1)
  eidx = jnp.stack([srcp.reshape(NW, CHI, K),
                    dstp.reshape(NW, CHI, K)], axis=2)
  zrows = jnp.zeros((RPT, D), jnp.float32)
  b0 = (bl0 + br0).reshape(1, D)
  b1 = (bl1 + br1).reshape(1, D)

  P0 = _propagate(x, eidx, zrows)
  h = _layer_tc(x, P0, Wl0, Wr0, b0)
  P1 = _propagate(h, eidx, zrows)
  return _final_tc(h, P1, Wl1, Wr1, b1, Wp1, bp1.reshape(1, D),
                   Wp2, bp2.reshape(1, D))


# speedup vs baseline: 1.0020x; 1.0020x over previous
import functools

import jax
import jax.numpy as jnp
from jax import lax
from jax.experimental import pallas as pl
from jax.experimental.pallas import tpu as pltpu
from jax.experimental.pallas import tpu_sc as plsc

N = 10000
D = 128
E = 320000
NC = 2
NS = 16
NW = NC * NS
EPT = E // NW
K = 128
CHI = 79
EPTP = CHI * K
NB = 2
NI = 4
RPT = 640
N_PAD = RPT * NS


def _propagate(x, eidx, zrows):
  mesh = plsc.VectorSubcoreMesh(core_axis_name="c", subcore_axis_name="s")

  @functools.partial(
      pl.kernel,
      out_type=jax.ShapeDtypeStruct((NC, N_PAD, D), jnp.float32),
      mesh=mesh,
      scratch_types=[
          pltpu.VMEM((NI, 2, K), jnp.int32),
          pltpu.VMEM((NB, K, D), jnp.float32),
          pltpu.VMEM_SHARED((N_PAD, D), jnp.float32),
          pltpu.SemaphoreType.DMA((NI,)),
          pltpu.SemaphoreType.DMA((NB,)),
      ],
  )
  def prop(x_hbm, e_hbm, z_hbm, out_hbm, idx_v, rows_v, acc,
           sem_i, sem_g):
    cid = lax.axis_index("c")
    sid = lax.axis_index("s")
    wid = sid * NC + cid
    pltpu.sync_copy(z_hbm, acc.at[pl.ds(sid * RPT, RPT)])

    def idx_fetch(j, si):
      pltpu.async_copy(e_hbm.at[wid, j], idx_v.at[si], sem_i.at[si])

    def idx_wait(j, si):
      pltpu.make_async_copy(e_hbm.at[wid, j], idx_v.at[si],
                            sem_i.at[si]).wait()

    def gather(j, b, si):
      pltpu.async_copy(x_hbm.at[idx_v.at[si, 0]], rows_v.at[b],
                       sem_g.at[b])

    def visit(j, b, si, do_idx=True, do_gather=True):
      bn = (b + 1) % NB
      pltpu.make_async_copy(x_hbm.at[idx_v.at[si, 0]], rows_v.at[b],
                            sem_g.at[b]).wait()
      if do_gather:
        jn = j + 1
        idx_wait(jn, jn % NI)
        gather(jn, bn, jn % NI)
      pltpu.sync_copy(rows_v.at[b], acc.at[idx_v.at[si, 1]], add=True)
      if do_idx:
        idx_fetch(j + NI, si)

    plsc.subcore_barrier()
    for si in range(NI):
      idx_fetch(si, si)
    idx_wait(0, 0)
    gather(0, 0, 0)
    for j in range(NI):
      visit(j, j % NB, j % NI,
            do_idx=j + NI < CHI, do_gather=j + 1 < CHI)

    G = CHI // NI
    R = CHI - G * NI

    def round_body(g, carry):
      for v in range(NI):
        visit(g * NI + v, v % NB, v % NI)
      return carry

    lax.fori_loop(1, G - 1, round_body, 0)
    for v in range(NI):
      j = (G - 1) * NI + v
      visit(j, v % NB, v % NI,
            do_idx=j + NI < CHI, do_gather=j + 1 < CHI)
    for v in range(R):
      j = G * NI + v
      visit(j, j % NB, j % NI,
            do_idx=j + NI < CHI, do_gather=j + 1 < CHI)
    plsc.subcore_barrier()
    pltpu.sync_copy(acc.at[pl.ds(sid * RPT, RPT)],
                    out_hbm.at[cid, pl.ds(sid * RPT, RPT)])

  return prop(x, eidx, zrows)


def _dotT(a, w):
  return lax.dot_general(a, w, (((1,), (1,)), ((), ())),
                         preferred_element_type=jnp.float32)


def _layer_body(x_ref, pa_ref, pb_ref, wl_ref, wr_ref, b_ref, o_ref):
  p = pa_ref[0] + pb_ref[0]
  h = _dotT(x_ref[...], wl_ref[...]) + _dotT(p, wr_ref[...]) + b_ref[...]
  nrm = jnp.sqrt(jnp.sum(h * h, axis=1, keepdims=True))
  h = h / jnp.maximum(nrm, 1e-12)
  o_ref[...] = jnp.maximum(h, 0.0)


def _final_body(x_ref, pa_ref, pb_ref, wl_ref, wr_ref, b_ref,
                wp1_ref, bp1_ref, wp2_ref, bp2_ref, o_ref):
  p = pa_ref[0] + pb_ref[0]
  h = _dotT(x_ref[...], wl_ref[...]) + _dotT(p, wr_ref[...]) + b_ref[...]
  nrm = jnp.sqrt(jnp.sum(h * h, axis=1, keepdims=True))
  h = h / jnp.maximum(nrm, 1e-12)
  h = jnp.maximum(h, 0.0)
  h = _dotT(h, wp1_ref[...]) + bp1_ref[...]
  h = _dotT(h, wp2_ref[...]) + bp2_ref[...]
  m = jnp.max(h, axis=1, keepdims=True)
  lse = m + jnp.log(jnp.sum(jnp.exp(h - m), axis=1, keepdims=True))
  o_ref[...] = h - lse


BR = 1000

_row_spec = pl.BlockSpec((BR, D), lambda i: (i, 0))
_p0_spec = pl.BlockSpec((1, BR, D), lambda i: (0, i, 0))
_p1_spec = pl.BlockSpec((1, BR, D), lambda i: (1, i, 0))
_w_spec = pl.BlockSpec((D, D), lambda i: (0, 0))
_b_spec = pl.BlockSpec((1, D), lambda i: (0, 0))


def _layer_tc(x, P, Wl, Wr, b):
  return pl.pallas_call(
      _layer_body,
      grid=(N // BR,),
      in_specs=[_row_spec, _p0_spec, _p1_spec, _w_spec, _w_spec, _b_spec],
      out_specs=_row_spec,
      out_shape=jax.ShapeDtypeStruct((N, D), jnp.float32),
  )(x, P, P, Wl, Wr, b)


def _final_tc(h, P, Wl, Wr, b, Wp1, bp1, Wp2, bp2):
  return pl.pallas_call(
      _final_body,
      grid=(N // BR,),
      in_specs=[_row_spec, _p0_spec, _p1_spec, _w_spec, _w_spec, _b_spec,
                _w_spec, _b_spec, _w_spec, _b_spec],
      out_specs=_row_spec,
      out_shape=jax.ShapeDtypeStruct((N, D), jnp.float32),
  )(h, P, P, Wl, Wr, b, Wp1, bp1, Wp2, bp2)


def kernel(x, edge_index, batch, Wl0, bl0, Wr0, br0, Wl1, bl1, Wr1, br1,
           Wp1, bp1, Wp2, bp2):
  del batch
  ei = edge_index.reshape(2, NW, EPT)
  pad = ((0, 0), (0, EPTP - EPT))
  srcp = jnp.pad(ei[0], pad, constant_values=0)
  dstp = jnp.pad(ei[1], pad, constant_values=N_PAD - 1)
  eidx = jnp.stack([srcp.reshape(NW, CHI, K),
                    dstp.reshape(NW, CHI, K)], axis=2)
  zrows = jnp.zeros((RPT, D), jnp.float32)
  b0 = (bl0 + br0).reshape(1, D)
  b1 = (bl1 + br1).reshape(1, D)

  P0 = _propagate(x, eidx, zrows)
  h = _layer_tc(x, P0, Wl0, Wr0, b0)
  P1 = _propagate(h, eidx, zrows)
  return _final_tc(h, P1, Wl1, Wr1, b1, Wp1, bp1.reshape(1, D),
                   Wp2, bp2.reshape(1, D))


# speedup vs baseline: 2.0877x; 2.0834x over previous
import functools

import jax
import jax.numpy as jnp
from jax import lax
from jax.experimental import pallas as pl
from jax.experimental.pallas import tpu as pltpu
from jax.experimental.pallas import tpu_sc as plsc

N = 10000
D = 128
E = 320000
NC = 2
NS = 16
NW = NC * NS
EPT = E // NW
K = 80
CHI = EPT // K
NB = 3
NI = 6
RPT = 640
N_PAD = RPT * NS


def _propagate(x, eidx, zrows):
  mesh = plsc.VectorSubcoreMesh(core_axis_name="c", subcore_axis_name="s")

  @functools.partial(
      pl.kernel,
      out_type=jax.ShapeDtypeStruct((NC, N_PAD, D), jnp.float32),
      mesh=mesh,
      scratch_types=[
          pltpu.VMEM((NI, 2, K), jnp.int32),
          pltpu.VMEM((NB, K, D), jnp.float32),
          pltpu.VMEM_SHARED((N_PAD, D), jnp.float32),
          pltpu.SemaphoreType.DMA((NI,)),
          pltpu.SemaphoreType.DMA((NB,)),
      ],
  )
  def prop(x_hbm, e_hbm, z_hbm, out_hbm, idx_v, rows_v, acc,
           sem_i, sem_g):
    cid = lax.axis_index("c")
    sid = lax.axis_index("s")
    wid = sid * NC + cid
    pltpu.sync_copy(z_hbm, acc.at[pl.ds(sid * RPT, RPT)])

    def idx_fetch(j, si):
      pltpu.async_copy(e_hbm.at[wid, j], idx_v.at[si], sem_i.at[si])

    def idx_wait(j, si):
      pltpu.make_async_copy(e_hbm.at[wid, j], idx_v.at[si],
                            sem_i.at[si]).wait()

    def gather(j, b, si):
      pltpu.async_copy(x_hbm.at[idx_v.at[si, 0]], rows_v.at[b],
                       sem_g.at[b])

    def visit(j, v, do_idx=True, do_gather=True):
      b, si = v % NB, v % NI
      pltpu.make_async_copy(x_hbm.at[idx_v.at[si, 0]], rows_v.at[b],
                            sem_g.at[b]).wait()
      if do_gather:
        jn = j + NB - 1
        vn = v + NB - 1
        idx_wait(jn, vn % NI)
        gather(jn, vn % NB, vn % NI)
      pltpu.sync_copy(rows_v.at[b], acc.at[idx_v.at[si, 1]], add=True)
      if do_idx:
        idx_fetch(j + NI, si)

    plsc.subcore_barrier()
    for si in range(NI):
      idx_fetch(si, si)
    for b in range(NB - 1):
      idx_wait(b, b)
      gather(b, b, b)
    for j in range(NI):
      visit(j, j, do_idx=j + NI < CHI, do_gather=j + NB - 1 < CHI)

    G = CHI // NI
    R = CHI - G * NI

    def round_body(g, carry):
      for v in range(NI):
        visit(g * NI + v, v)
      return carry

    lax.fori_loop(1, G - 1, round_body, 0)
    for v in range(NI):
      j = (G - 1) * NI + v
      visit(j, v, do_idx=j + NI < CHI, do_gather=j + NB - 1 < CHI)
    for v in range(R):
      j = G * NI + v
      visit(j, v, do_idx=j + NI < CHI, do_gather=j + NB - 1 < CHI)
    plsc.subcore_barrier()
    pltpu.sync_copy(acc.at[pl.ds(sid * RPT, RPT)],
                    out_hbm.at[cid, pl.ds(sid * RPT, RPT)])

  return prop(x, eidx, zrows)


def _dotT(a, w):
  return lax.dot_general(a, w, (((1,), (1,)), ((), ())),
                         preferred_element_type=jnp.float32)


def _layer_body(x_ref, pa_ref, pb_ref, wl_ref, wr_ref, b_ref, o_ref):
  p = pa_ref[0] + pb_ref[0]
  h = _dotT(x_ref[...], wl_ref[...]) + _dotT(p, wr_ref[...]) + b_ref[...]
  nrm = jnp.sqrt(jnp.sum(h * h, axis=1, keepdims=True))
  h = h / jnp.maximum(nrm, 1e-12)
  o_ref[...] = jnp.maximum(h, 0.0)


def _final_body(x_ref, pa_ref, pb_ref, wl_ref, wr_ref, b_ref,
                wp1_ref, bp1_ref, wp2_ref, bp2_ref, o_ref):
  p = pa_ref[0] + pb_ref[0]
  h = _dotT(x_ref[...], wl_ref[...]) + _dotT(p, wr_ref[...]) + b_ref[...]
  nrm = jnp.sqrt(jnp.sum(h * h, axis=1, keepdims=True))
  h = h / jnp.maximum(nrm, 1e-12)
  h = jnp.maximum(h, 0.0)
  h = _dotT(h, wp1_ref[...]) + bp1_ref[...]
  h = _dotT(h, wp2_ref[...]) + bp2_ref[...]
  m = jnp.max(h, axis=1, keepdims=True)
  lse = m + jnp.log(jnp.sum(jnp.exp(h - m), axis=1, keepdims=True))
  o_ref[...] = h - lse


BR = 1000

_row_spec = pl.BlockSpec((BR, D), lambda i: (i, 0))
_p0_spec = pl.BlockSpec((1, BR, D), lambda i: (0, i, 0))
_p1_spec = pl.BlockSpec((1, BR, D), lambda i: (1, i, 0))
_w_spec = pl.BlockSpec((D, D), lambda i: (0, 0))
_b_spec = pl.BlockSpec((1, D), lambda i: (0, 0))


def _layer_tc(x, P, Wl, Wr, b):
  return pl.pallas_call(
      _layer_body,
      grid=(N // BR,),
      in_specs=[_row_spec, _p0_spec, _p1_spec, _w_spec, _w_spec, _b_spec],
      out_specs=_row_spec,
      out_shape=jax.ShapeDtypeStruct((N, D), jnp.float32),
  )(x, P, P, Wl, Wr, b)


def _final_tc(h, P, Wl, Wr, b, Wp1, bp1, Wp2, bp2):
  return pl.pallas_call(
      _final_body,
      grid=(N // BR,),
      in_specs=[_row_spec, _p0_spec, _p1_spec, _w_spec, _w_spec, _b_spec,
                _w_spec, _b_spec, _w_spec, _b_spec],
      out_specs=_row_spec,
      out_shape=jax.ShapeDtypeStruct((N, D), jnp.float32),
  )(h, P, P, Wl, Wr, b, Wp1, bp1, Wp2, bp2)


def kernel(x, edge_index, batch, Wl0, bl0, Wr0, br0, Wl1, bl1, Wr1, br1,
           Wp1, bp1, Wp2, bp2):
  del batch
  ei = edge_index.reshape(2, NW, EPT)
  eidx = jnp.stack([ei[0].reshape(NW, CHI, K),
                    ei[1].reshape(NW, CHI, K)], axis=2)
  zrows = jnp.zeros((RPT, D), jnp.float32)
  b0 = (bl0 + br0).reshape(1, D)
  b1 = (bl1 + br1).reshape(1, D)

  P0 = _propagate(x, eidx, zrows)
  h = _layer_tc(x, P0, Wl0, Wr0, b0)
  P1 = _propagate(h, eidx, zrows)
  return _final_tc(h, P1, Wl1, Wr1, b1, Wp1, bp1.reshape(1, D),
                   Wp2, bp2.reshape(1, D))


# speedup vs baseline: 2.2052x; 1.0563x over previous
import functools

import jax
import jax.numpy as jnp
from jax import lax
from jax.experimental import pallas as pl
from jax.experimental.pallas import tpu as pltpu
from jax.experimental.pallas import tpu_sc as plsc

N = 10000
D = 128
E = 320000
NC = 2
NS = 16
NW = NC * NS
EPT = E // NW
K = 100
CHI = EPT // K
NB = 3
NI = 6
RPT = 640
N_PAD = RPT * NS


def _propagate(x, eidx, zrows):
  mesh = plsc.VectorSubcoreMesh(core_axis_name="c", subcore_axis_name="s")

  @functools.partial(
      pl.kernel,
      out_type=jax.ShapeDtypeStruct((NC, N_PAD, D), jnp.float32),
      mesh=mesh,
      scratch_types=[
          pltpu.VMEM((NI, 2, K), jnp.int32),
          pltpu.VMEM((NB, K, D), jnp.float32),
          pltpu.VMEM_SHARED((N_PAD, D), jnp.float32),
          pltpu.SemaphoreType.DMA((NI,)),
          pltpu.SemaphoreType.DMA((NB,)),
      ],
  )
  def prop(x_hbm, e_hbm, z_hbm, out_hbm, idx_v, rows_v, acc,
           sem_i, sem_g):
    cid = lax.axis_index("c")
    sid = lax.axis_index("s")
    wid = sid * NC + cid
    pltpu.sync_copy(z_hbm, acc.at[pl.ds(sid * RPT, RPT)])

    def idx_fetch(j, si):
      pltpu.async_copy(e_hbm.at[wid, j], idx_v.at[si], sem_i.at[si])

    def idx_wait(j, si):
      pltpu.make_async_copy(e_hbm.at[wid, j], idx_v.at[si],
                            sem_i.at[si]).wait()

    def gather(j, b, si):
      pltpu.async_copy(x_hbm.at[idx_v.at[si, 0]], rows_v.at[b],
                       sem_g.at[b])

    def visit(j, v, do_idx=True, do_gather=True):
      b, si = v % NB, v % NI
      pltpu.make_async_copy(x_hbm.at[idx_v.at[si, 0]], rows_v.at[b],
                            sem_g.at[b]).wait()
      if do_gather:
        jn = j + NB - 1
        vn = v + NB - 1
        idx_wait(jn, vn % NI)
        gather(jn, vn % NB, vn % NI)
      pltpu.sync_copy(rows_v.at[b], acc.at[idx_v.at[si, 1]], add=True)
      if do_idx:
        idx_fetch(j + NI, si)

    plsc.subcore_barrier()
    for si in range(NI):
      idx_fetch(si, si)
    for b in range(NB - 1):
      idx_wait(b, b)
      gather(b, b, b)
    for j in range(NI):
      visit(j, j, do_idx=j + NI < CHI, do_gather=j + NB - 1 < CHI)

    G = CHI // NI
    R = CHI - G * NI

    def round_body(g, carry):
      for v in range(NI):
        visit(g * NI + v, v)
      return carry

    lax.fori_loop(1, G - 1, round_body, 0)
    for v in range(NI):
      j = (G - 1) * NI + v
      visit(j, v, do_idx=j + NI < CHI, do_gather=j + NB - 1 < CHI)
    for v in range(R):
      j = G * NI + v
      visit(j, v, do_idx=j + NI < CHI, do_gather=j + NB - 1 < CHI)
    plsc.subcore_barrier()
    pltpu.sync_copy(acc.at[pl.ds(sid * RPT, RPT)],
                    out_hbm.at[cid, pl.ds(sid * RPT, RPT)])

  return prop(x, eidx, zrows)


def _dotT(a, w):
  return lax.dot_general(a, w, (((1,), (1,)), ((), ())),
                         preferred_element_type=jnp.float32)


def _layer_body(x_ref, pa_ref, pb_ref, wl_ref, wr_ref, b_ref, o_ref):
  p = pa_ref[0] + pb_ref[0]
  h = _dotT(x_ref[...], wl_ref[...]) + _dotT(p, wr_ref[...]) + b_ref[...]
  nrm = jnp.sqrt(jnp.sum(h * h, axis=1, keepdims=True))
  h = h / jnp.maximum(nrm, 1e-12)
  o_ref[...] = jnp.maximum(h, 0.0)


def _final_body(x_ref, pa_ref, pb_ref, wl_ref, wr_ref, b_ref,
                wp1_ref, bp1_ref, wp2_ref, bp2_ref, o_ref):
  p = pa_ref[0] + pb_ref[0]
  h = _dotT(x_ref[...], wl_ref[...]) + _dotT(p, wr_ref[...]) + b_ref[...]
  nrm = jnp.sqrt(jnp.sum(h * h, axis=1, keepdims=True))
  h = h / jnp.maximum(nrm, 1e-12)
  h = jnp.maximum(h, 0.0)
  h = _dotT(h, wp1_ref[...]) + bp1_ref[...]
  h = _dotT(h, wp2_ref[...]) + bp2_ref[...]
  m = jnp.max(h, axis=1, keepdims=True)
  lse = m + jnp.log(jnp.sum(jnp.exp(h - m), axis=1, keepdims=True))
  o_ref[...] = h - lse


BR = 1000

_row_spec = pl.BlockSpec((BR, D), lambda i: (i, 0))
_p0_spec = pl.BlockSpec((1, BR, D), lambda i: (0, i, 0))
_p1_spec = pl.BlockSpec((1, BR, D), lambda i: (1, i, 0))
_w_spec = pl.BlockSpec((D, D), lambda i: (0, 0))
_b_spec = pl.BlockSpec((1, D), lambda i: (0, 0))


def _layer_tc(x, P, Wl, Wr, b):
  return pl.pallas_call(
      _layer_body,
      grid=(N // BR,),
      in_specs=[_row_spec, _p0_spec, _p1_spec, _w_spec, _w_spec, _b_spec],
      out_specs=_row_spec,
      out_shape=jax.ShapeDtypeStruct((N, D), jnp.float32),
  )(x, P, P, Wl, Wr, b)


def _final_tc(h, P, Wl, Wr, b, Wp1, bp1, Wp2, bp2):
  return pl.pallas_call(
      _final_body,
      grid=(N // BR,),
      in_specs=[_row_spec, _p0_spec, _p1_spec, _w_spec, _w_spec, _b_spec,
                _w_spec, _b_spec, _w_spec, _b_spec],
      out_specs=_row_spec,
      out_shape=jax.ShapeDtypeStruct((N, D), jnp.float32),
  )(h, P, P, Wl, Wr, b, Wp1, bp1, Wp2, bp2)


def kernel(x, edge_index, batch, Wl0, bl0, Wr0, br0, Wl1, bl1, Wr1, br1,
           Wp1, bp1, Wp2, bp2):
  del batch
  ei = edge_index.reshape(2, NW, EPT)
  eidx = jnp.stack([ei[0].reshape(NW, CHI, K),
                    ei[1].reshape(NW, CHI, K)], axis=2)
  zrows = jnp.zeros((RPT, D), jnp.float32)
  b0 = (bl0 + br0).reshape(1, D)
  b1 = (bl1 + br1).reshape(1, D)

  P0 = _propagate(x, eidx, zrows)
  h = _layer_tc(x, P0, Wl0, Wr0, b0)
  P1 = _propagate(h, eidx, zrows)
  return _final_tc(h, P1, Wl1, Wr1, b1, Wp1, bp1.reshape(1, D),
                   Wp2, bp2.reshape(1, D))
